# Initial kernel scaffold; baseline (speedup 1.0000x reference)
import functools

import jax
import jax.numpy as jnp
from jax import lax
from jax.experimental import pallas as pl
from jax.experimental.pallas import tpu as pltpu
from jax.experimental.pallas import tpu_sc as plsc

NC, NS = 2, 16
NW = NC * NS
ROWS_PER_DMA = 128
NBUF = 8
DELAY = 4


def _emb_body(idx_hbm, table_hbm, out_hbm, idx_v, rows_v, gsem, wsem, *, g_per_w):
    wid = lax.axis_index("s") * NC + lax.axis_index("c")
    row0 = wid * g_per_w
    pltpu.sync_copy(idx_hbm.at[pl.ds(row0, g_per_w)], idx_v)

    def _wait_gather(s):
        pltpu.make_async_copy(
            table_hbm.at[pl.ds(0, ROWS_PER_DMA)], rows_v.at[s], gsem.at[s]
        ).wait()

    def _wait_write(s):
        pltpu.make_async_copy(rows_v.at[s], out_hbm.at[row0], wsem.at[s]).wait()

    @pl.loop(0, g_per_w, step=NBUF)
    def _(g0):
        for b in range(NBUF):
            g = g0 + b
            s = b

            @pl.when(g >= NBUF)
            def _():
                _wait_write(s)

            pltpu.async_copy(
                table_hbm.at[idx_v.at[g]], rows_v.at[s], gsem.at[s]
            )

            d = g - DELAY
            sd = (b - DELAY) % NBUF

            @pl.when(d >= 0)
            def _():
                _wait_gather(sd)
                pltpu.async_copy(rows_v.at[sd], out_hbm.at[row0 + d], wsem.at[sd])

    for e in range(DELAY):
        d = g_per_w - DELAY + e
        sd = d % NBUF
        _wait_gather(sd)
        pltpu.async_copy(rows_v.at[sd], out_hbm.at[row0 + d], wsem.at[sd])
    for s in range(NBUF):
        _wait_write(s)


def kernel(inputs, weight):
    bsz, hist = inputs.shape
    vocab, dim = weight.shape
    total = bsz * hist
    assert total % (ROWS_PER_DMA * NW) == 0
    n_chunks = total // ROWS_PER_DMA
    g_per_w = n_chunks // NW
    assert g_per_w % NBUF == 0

    idx = inputs.reshape(n_chunks, ROWS_PER_DMA).astype(jnp.int32)

    run = pl.kernel(
        functools.partial(_emb_body, g_per_w=g_per_w),
        out_type=jax.ShapeDtypeStruct((n_chunks, ROWS_PER_DMA, dim), jnp.float32),
        mesh=plsc.VectorSubcoreMesh(
            core_axis_name="c", subcore_axis_name="s",
            num_cores=NC, num_subcores=NS,
        ),
        scratch_types=[
            pltpu.VMEM((g_per_w, ROWS_PER_DMA), jnp.int32),
            pltpu.VMEM((NBUF, ROWS_PER_DMA, dim), jnp.float32),
            pltpu.SemaphoreType.DMA((NBUF,)),
            pltpu.SemaphoreType.DMA((NBUF,)),
        ],
    )
    out = run(idx, weight)
    return out.reshape(bsz, hist, dim)


# speedup vs baseline: 1.8748x; 1.8748x over previous
import functools

import jax
import jax.numpy as jnp
from jax import lax
from jax.experimental import pallas as pl
from jax.experimental.pallas import tpu as pltpu
from jax.experimental.pallas import tpu_sc as plsc

NC, NS = 2, 16
NW = NC * NS
ROWS_PER_DMA = 128
NBUF = 8
DELAY = 4


def _emb_body(idx_hbm, table_hbm, out_hbm, idx_v, rows_v, gsem, wsem, *, g_per_w):
    wid = lax.axis_index("s") * NC + lax.axis_index("c")
    row0 = wid * g_per_w
    pltpu.sync_copy(idx_hbm.at[pl.ds(row0, g_per_w)], idx_v)

    def _wait_gather(s):
        pltpu.make_async_copy(
            table_hbm.at[pl.ds(0, ROWS_PER_DMA)], rows_v.at[s], gsem.at[s]
        ).wait()

    def _wait_write(s):
        pltpu.make_async_copy(rows_v.at[s], out_hbm.at[row0], wsem.at[s]).wait()

    @pl.loop(0, g_per_w, step=NBUF)
    def _(g0):
        for b in range(NBUF):
            g = g0 + b
            s = b

            @pl.when(g >= NBUF)
            def _():
                _wait_write(s)

            pltpu.async_copy(
                table_hbm.at[idx_v.at[g]], rows_v.at[s], gsem.at[s]
            )

            d = g - DELAY
            sd = (b - DELAY) % NBUF

            @pl.when(d >= 0)
            def _():
                _wait_gather(sd)
                pltpu.async_copy(rows_v.at[sd], out_hbm.at[row0 + d], wsem.at[sd])

    for e in range(DELAY):
        d = g_per_w - DELAY + e
        sd = d % NBUF
        _wait_gather(sd)
        pltpu.async_copy(rows_v.at[sd], out_hbm.at[row0 + d], wsem.at[sd])
    for s in range(NBUF):
        _wait_write(s)


def kernel(inputs, weight):
    bsz, hist = inputs.shape
    vocab, dim = weight.shape
    total = bsz * hist
    assert total % (ROWS_PER_DMA * NW) == 0
    n_chunks = total // ROWS_PER_DMA
    g_per_w = n_chunks // NW
    assert g_per_w % NBUF == 0

    idx = inputs.reshape(n_chunks, ROWS_PER_DMA).astype(jnp.int32)

    run = pl.kernel(
        functools.partial(_emb_body, g_per_w=g_per_w),
        out_type=jax.ShapeDtypeStruct((n_chunks, ROWS_PER_DMA, dim), jnp.float32),
        mesh=plsc.VectorSubcoreMesh(
            core_axis_name="c", subcore_axis_name="s",
            num_cores=NC, num_subcores=NS,
        ),
        scratch_types=[
            pltpu.VMEM((g_per_w, ROWS_PER_DMA), jnp.int32),
            pltpu.VMEM((NBUF, ROWS_PER_DMA, dim), jnp.float32),
            pltpu.SemaphoreType.DMA((NBUF,)),
            pltpu.SemaphoreType.DMA((NBUF,)),
        ],
        compiler_params=pltpu.CompilerParams(use_tc_tiling_on_sc=False),
    )
    out = run(idx, weight)
    return out.reshape(bsz, hist, dim)
